# baseline (speedup 1.0000x reference)
import functools

import jax
import jax.numpy as jnp
from jax import lax
from jax.experimental import pallas as pl
from jax.experimental.pallas import tpu as pltpu
from jax.experimental.pallas import tpu_sc as plsc

_V, _E, _H, _B, _L = 32000, 256, 512, 16, 512
_NP = 2
_LP = _L // _NP
_NH = _B * _LP
_TB = 32


@functools.cache
def _make_sc_gather():
    info = plsc.get_sparse_core_info()
    nw = info.num_cores * info.num_subcores
    rows_per_w = _NH // nw
    mesh = plsc.VectorSubcoreMesh(core_axis_name="c", subcore_axis_name="s")

    @functools.partial(
        pl.kernel,
        mesh=mesh,
        out_type=jax.ShapeDtypeStruct((_NH, _E), jnp.float32),
        scratch_types=[
            pltpu.VMEM((rows_per_w,), jnp.int32),
            pltpu.VMEM((rows_per_w, _E), jnp.float32),
            pltpu.SemaphoreType.DMA,
        ],
    )
    def gather_k(table_hbm, idx_hbm, out_hbm, idx_v, rows_v, sem):
        wid = lax.axis_index("s") * info.num_cores + lax.axis_index("c")
        base = wid * rows_per_w
        pltpu.sync_copy(idx_hbm.at[pl.ds(base, rows_per_w)], idx_v)
        pltpu.async_copy(table_hbm.at[idx_v], rows_v, sem).wait()
        pltpu.sync_copy(rows_v, out_hbm.at[pl.ds(base, rows_per_w)])

    return gather_k


def _make_gru_phase(phase):
    t0 = phase * _LP
    nblk = _LP // _TB

    def body(ml_ref, len_ref, x_ref, wih_ref, bih_ref, whh_ref, bhh_ref,
             hin_ref, y_ref, hout_ref, h_ref, gi_ref):
        tb = pl.program_id(0)

        @pl.when(tb == 0)
        def _():
            h_ref[...] = hin_ref[...] if phase else jnp.zeros_like(h_ref)

        @pl.when(t0 + tb * _TB < ml_ref[0])
        def _():
            gi_ref[...] = (
                jnp.dot(
                    x_ref[...].astype(jnp.bfloat16),
                    wih_ref[...],
                    preferred_element_type=jnp.float32,
                )
                + bih_ref[...]
            )
            h = h_ref[...]
            lens = len_ref[...]
            for j in range(_TB):
                t = t0 + tb * _TB + j
                gh = (
                    jnp.dot(
                        h.astype(jnp.bfloat16),
                        whh_ref[...],
                        preferred_element_type=jnp.float32,
                    )
                    + bhh_ref[...]
                )
                gi = gi_ref[j * _B : (j + 1) * _B, :]
                r = jax.nn.sigmoid(gi[:, :_H] + gh[:, :_H])
                z = jax.nn.sigmoid(gi[:, _H : 2 * _H] + gh[:, _H : 2 * _H])
                n = jnp.tanh(gi[:, 2 * _H :] + r * gh[:, 2 * _H :])
                h_new = (1.0 - z) * n + z * h
                m = t < lens
                h = jnp.where(m, h_new, h)
                y_ref[:, j] = jnp.where(m, h_new, 0.0)
            h_ref[...] = h

        @pl.when(t0 + tb * _TB >= ml_ref[0])
        def _():
            y_ref[...] = jnp.zeros_like(y_ref)

        hout_ref[...] = h_ref[...]

    def run(maxlen, lens2d, x_tm, wih, bih, whh, bhh, h_in):
        grid_spec = pltpu.PrefetchScalarGridSpec(
            num_scalar_prefetch=1,
            grid=(nblk,),
            in_specs=[
                pl.BlockSpec((_B, 1), lambda i, ml: (0, 0)),
                pl.BlockSpec((_TB * _B, _E), lambda i, ml: (i, 0)),
                pl.BlockSpec((_E, 3 * _H), lambda i, ml: (0, 0)),
                pl.BlockSpec((1, 3 * _H), lambda i, ml: (0, 0)),
                pl.BlockSpec((_H, 3 * _H), lambda i, ml: (0, 0)),
                pl.BlockSpec((1, 3 * _H), lambda i, ml: (0, 0)),
                pl.BlockSpec((_B, _H), lambda i, ml: (0, 0)),
            ],
            out_specs=[
                pl.BlockSpec((_B, _TB, _H), lambda i, ml: (0, i, 0)),
                pl.BlockSpec((_B, _H), lambda i, ml: (0, 0)),
            ],
            scratch_shapes=[
                pltpu.VMEM((_B, _H), jnp.float32),
                pltpu.VMEM((_TB * _B, 3 * _H), jnp.float32),
            ],
        )
        return pl.pallas_call(
            body,
            grid_spec=grid_spec,
            out_shape=[
                jax.ShapeDtypeStruct((_B, _LP, _H), jnp.float32),
                jax.ShapeDtypeStruct((_B, _H), jnp.float32),
            ],
        )(maxlen, lens2d, x_tm, wih, bih, whh, bhh, h_in)

    return run


def kernel(tokens, seq_lengths, embed_table, W_ih, W_hh, b_ih, b_hh):
    idx = tokens.T.reshape(_L * _B).astype(jnp.int32)
    gather = _make_sc_gather()
    x0 = gather(embed_table, idx[:_NH])
    x1 = gather(embed_table, idx[_NH:])
    lens = seq_lengths.astype(jnp.int32)
    maxlen = jnp.max(lens).reshape(1)
    lens2d = lens.reshape(_B, 1)
    wih = W_ih.T.astype(jnp.bfloat16)
    bih = b_ih.reshape(1, 3 * _H)
    whh = W_hh.T.astype(jnp.bfloat16)
    bhh = b_hh.reshape(1, 3 * _H)
    h0 = jnp.zeros((_B, _H), jnp.float32)
    y0, h1 = _make_gru_phase(0)(maxlen, lens2d, x0, wih, bih, whh, bhh, h0)
    y1, _ = _make_gru_phase(1)(maxlen, lens2d, x1, wih, bih, whh, bhh, h1)
    return jnp.concatenate([y0, y1], axis=1)


# speedup vs baseline: 1.0782x; 1.0782x over previous
import functools

import jax
import jax.numpy as jnp
from jax import lax
from jax.experimental import pallas as pl
from jax.experimental.pallas import tpu as pltpu
from jax.experimental.pallas import tpu_sc as plsc

_V, _E, _H, _B, _L = 32000, 256, 512, 16, 512
_N = _B * _L


@functools.cache
def _make_sc_gather():
    info = plsc.get_sparse_core_info()
    nw = info.num_cores * info.num_subcores
    rows_per_w = _N // nw
    mesh = plsc.VectorSubcoreMesh(core_axis_name="c", subcore_axis_name="s")

    @functools.partial(
        pl.kernel,
        mesh=mesh,
        out_type=jax.ShapeDtypeStruct((_N, _E), jnp.float32),
        scratch_types=[
            pltpu.VMEM((rows_per_w,), jnp.int32),
            pltpu.VMEM((rows_per_w, _E), jnp.float32),
            pltpu.SemaphoreType.DMA,
        ],
    )
    def gather_k(table_hbm, idx_hbm, out_hbm, idx_v, rows_v, sem):
        wid = lax.axis_index("s") * info.num_cores + lax.axis_index("c")
        base = wid * rows_per_w
        pltpu.sync_copy(idx_hbm.at[pl.ds(base, rows_per_w)], idx_v)
        pltpu.async_copy(table_hbm.at[idx_v], rows_v, sem).wait()
        pltpu.sync_copy(rows_v, out_hbm.at[pl.ds(base, rows_per_w)])

    return gather_k


_TB = 32


def _gru_body(ml_ref, len_ref, x_ref, wih_ref, bih_ref, whh_ref, bhh_ref,
              y_ref, h_ref, gi_ref):
    tb = pl.program_id(0)

    @pl.when(tb == 0)
    def _():
        h_ref[...] = jnp.zeros_like(h_ref)

    @pl.when(tb * _TB < ml_ref[0])
    def _():
        gi_ref[...] = (
            jnp.dot(
                x_ref[...].astype(jnp.bfloat16),
                wih_ref[...],
                preferred_element_type=jnp.float32,
            )
            + bih_ref[...]
        )

        h = h_ref[...]
        lens = len_ref[...]
        for j in range(_TB):
            t = tb * _TB + j
            gh = (
                jnp.dot(
                    h.astype(jnp.bfloat16),
                    whh_ref[...],
                    preferred_element_type=jnp.float32,
                )
                + bhh_ref[...]
            )
            gi = gi_ref[j * _B : (j + 1) * _B, :]
            r = jax.nn.sigmoid(gi[:, :_H] + gh[:, :_H])
            z = jax.nn.sigmoid(gi[:, _H : 2 * _H] + gh[:, _H : 2 * _H])
            n = jnp.tanh(gi[:, 2 * _H :] + r * gh[:, 2 * _H :])
            h_new = (1.0 - z) * n + z * h
            m = t < lens
            h = jnp.where(m, h_new, h)
            y_ref[:, j] = jnp.where(m, h_new, 0.0)
        h_ref[...] = h

    @pl.when(tb * _TB >= ml_ref[0])
    def _():
        y_ref[...] = jnp.zeros_like(y_ref)


def _gru_scan(maxlen, lens2d, x_tm, w_ih_t, bih2d, w_hh_t, bhh2d):
    grid_spec = pltpu.PrefetchScalarGridSpec(
        num_scalar_prefetch=1,
        grid=(_L // _TB,),
        in_specs=[
            pl.BlockSpec((_B, 1), lambda i, ml: (0, 0)),
            pl.BlockSpec((_TB * _B, _E), lambda i, ml: (i, 0)),
            pl.BlockSpec((_E, 3 * _H), lambda i, ml: (0, 0)),
            pl.BlockSpec((1, 3 * _H), lambda i, ml: (0, 0)),
            pl.BlockSpec((_H, 3 * _H), lambda i, ml: (0, 0)),
            pl.BlockSpec((1, 3 * _H), lambda i, ml: (0, 0)),
        ],
        out_specs=pl.BlockSpec((_B, _TB, _H), lambda i, ml: (0, i, 0)),
        scratch_shapes=[
            pltpu.VMEM((_B, _H), jnp.float32),
            pltpu.VMEM((_TB * _B, 3 * _H), jnp.float32),
        ],
    )
    return pl.pallas_call(
        _gru_body,
        grid_spec=grid_spec,
        out_shape=jax.ShapeDtypeStruct((_B, _L, _H), jnp.float32),
    )(maxlen, lens2d, x_tm, w_ih_t, bih2d, w_hh_t, bhh2d)


def kernel(tokens, seq_lengths, embed_table, W_ih, W_hh, b_ih, b_hh):
    idx = tokens.T.reshape(_N).astype(jnp.int32)
    x_tm = _make_sc_gather()(embed_table, idx)
    lens = seq_lengths.astype(jnp.int32)
    return _gru_scan(
        jnp.max(lens).reshape(1),
        lens.reshape(_B, 1),
        x_tm,
        W_ih.T.astype(jnp.bfloat16),
        b_ih.reshape(1, 3 * _H),
        W_hh.T.astype(jnp.bfloat16),
        b_hh.reshape(1, 3 * _H),
    )
